# baseline (speedup 1.0000x reference)
import functools

import jax
import jax.numpy as jnp
from jax import lax
from jax.experimental import pallas as pl
from jax.experimental.pallas import tpu as pltpu
from jax.experimental.pallas import tpu_sc as plsc

N = 10000
E = 320000
D_IN = 128
H = 256
HALF = 128

N_PAD = 10240

_MESH = plsc.VectorSubcoreMesh(core_axis_name="c", subcore_axis_name="s")

DEG_CHUNK = 80
DEG_PER_SUB = E // 32


NBUF = 5
DEG_T = DEG_PER_SUB // DEG_CHUNK
DEG_K = DEG_T // NBUF


@functools.partial(
    pl.kernel,
    out_type=jax.ShapeDtypeStruct((2, N_PAD), jnp.float32),
    mesh=_MESH,
    scratch_types=(
        [pltpu.VMEM((DEG_CHUNK,), jnp.int32) for _ in range(NBUF)]
        + [pltpu.VMEM((DEG_CHUNK,), jnp.float32)]
        + [pltpu.VMEM((640,), jnp.float32)]
        + [pltpu.SemaphoreType.DMA for _ in range(2 * NBUF)]
        + [pltpu.VMEM_SHARED((N_PAD,), jnp.float32)]
    ),
)
def _deg_kernel(dst_hbm, out_hbm, *rest):
    idx_v = rest[:NBUF]
    ones_v = rest[NBUF]
    zeros_v = rest[NBUF + 1]
    semI = rest[NBUF + 2:2 * NBUF + 2]
    semS = rest[2 * NBUF + 2:3 * NBUF + 2]
    acc_sh = rest[3 * NBUF + 2]

    c = lax.axis_index("c")
    s = lax.axis_index("s")
    base = (c * 16 + s) * DEG_PER_SUB

    def fill(i):
        zeros_v[pl.ds(i * 16, 16)] = jnp.zeros((16,), jnp.float32)
    pl.loop(0, 40)(fill)

    def fill1(i):
        ones_v[pl.ds(i * 16, 16)] = jnp.ones((16,), jnp.float32)
    pl.loop(0, DEG_CHUNK // 16)(fill1)

    pltpu.sync_copy(zeros_v, acc_sh.at[pl.ds(s * 640, 640)])
    plsc.subcore_barrier()

    def issue_idx(t, b):
        pltpu.async_copy(
            dst_hbm.at[pl.ds(base + t * DEG_CHUNK, DEG_CHUNK)], idx_v[b], semI[b])

    for b in range(NBUF):
        issue_idx(b, b)

    def outer(k):
        for b in range(NBUF):
            t = k * NBUF + b
            pltpu.make_async_copy(
                dst_hbm.at[pl.ds(base + t * DEG_CHUNK, DEG_CHUNK)], idx_v[b],
                semI[b]).wait()
            pltpu.async_copy(ones_v, acc_sh.at[idx_v[b]], semS[b], add=True)
            b2 = (b - 1) % NBUF

            @pl.when(jnp.logical_and(t >= 1, t + (NBUF - 1) <= DEG_T - 1))
            def _():
                pltpu.make_async_copy(
                    ones_v, acc_sh.at[idx_v[b2]], semS[b2]).wait()
                issue_idx(t + NBUF - 1, b2)
    pl.loop(0, DEG_K)(outer)

    for i in range(NBUF):
        pltpu.make_async_copy(ones_v, acc_sh.at[idx_v[i]], semS[i]).wait()

    plsc.subcore_barrier()
    pltpu.sync_copy(acc_sh.at[pl.ds(s * 640, 640)],
                    out_hbm.at[c].at[pl.ds(s * 640, 640)])


SC_CHUNK = 40
SC_PER_SUB = E // 16


RPS = N_PAD // 16
SC_T = SC_PER_SUB // SC_CHUNK
SC_K = SC_T // NBUF


@functools.partial(
    pl.kernel,
    out_type=jax.ShapeDtypeStruct((2, N_PAD, HALF), jnp.float32),
    mesh=_MESH,
    scratch_types=(
        [pltpu.VMEM((SC_CHUNK, HALF), jnp.float32) for _ in range(NBUF)]
        + [pltpu.VMEM((SC_CHUNK,), jnp.int32) for _ in range(NBUF)]
        + [pltpu.VMEM((SC_PER_SUB,), jnp.int32)]
        + [pltpu.SemaphoreType.DMA for _ in range(3 * NBUF)]
        + [pltpu.VMEM_SHARED((N_PAD, HALF), jnp.float32)]
    ),
)
def _scatter_kernel(g_hbm, src_hbm, dst_hbm, out_hbm, *rest):
    rows_v = rest[:NBUF]
    dst_v = rest[NBUF:2 * NBUF]
    src_all = rest[2 * NBUF]
    semI = rest[2 * NBUF + 1:3 * NBUF + 1]
    semG = rest[3 * NBUF + 1:4 * NBUF + 1]
    semS = rest[4 * NBUF + 1:5 * NBUF + 1]
    acc_sh = rest[5 * NBUF + 1]

    c = lax.axis_index("c")
    s = lax.axis_index("s")
    base = s * SC_PER_SUB

    pltpu.async_copy(src_hbm.at[pl.ds(base, SC_PER_SUB)], src_all, semG[0])

    def zrow(i):
        def zcol(j):
            rows_v[0][i, pl.ds(j * 16, 16)] = jnp.zeros((16,), jnp.float32)
        pl.loop(0, HALF // 16)(zcol)
    pl.loop(0, SC_CHUNK)(zrow)

    r0 = s * RPS

    def zacc(k):
        pltpu.async_copy(rows_v[0], acc_sh.at[pl.ds(r0 + k * SC_CHUNK, SC_CHUNK)],
                         semS[0])
    pl.loop(0, RPS // SC_CHUNK)(zacc)

    def zwait(k):
        pltpu.make_async_copy(
            rows_v[0], acc_sh.at[pl.ds(r0 + k * SC_CHUNK, SC_CHUNK)], semS[0]).wait()
    pl.loop(0, RPS // SC_CHUNK)(zwait)
    pltpu.make_async_copy(src_hbm.at[pl.ds(base, SC_PER_SUB)], src_all,
                          semG[0]).wait()

    plsc.subcore_barrier()

    def issue_chunk(t, b):
        pltpu.async_copy(
            dst_hbm.at[pl.ds(base + t * SC_CHUNK, SC_CHUNK)], dst_v[b], semI[b])
        pltpu.async_copy(
            g_hbm.at[c].at[src_all.at[pl.ds(t * SC_CHUNK, SC_CHUNK)]],
            rows_v[b], semG[b])

    for b in range(NBUF):
        issue_chunk(b, b)

    def outer(k):
        for b in range(NBUF):
            t = k * NBUF + b
            pltpu.make_async_copy(
                dst_hbm.at[pl.ds(base + t * SC_CHUNK, SC_CHUNK)], dst_v[b],
                semI[b]).wait()
            pltpu.make_async_copy(
                g_hbm.at[c].at[src_all.at[pl.ds(t * SC_CHUNK, SC_CHUNK)]],
                rows_v[b], semG[b]).wait()
            pltpu.async_copy(rows_v[b], acc_sh.at[dst_v[b]], semS[b], add=True)
            b2 = (b - 2) % NBUF

            @pl.when(jnp.logical_and(t >= 2, t + (NBUF - 2) <= SC_T - 1))
            def _():
                pltpu.make_async_copy(
                    rows_v[b2], acc_sh.at[dst_v[b2]], semS[b2]).wait()
                issue_chunk(t + NBUF - 2, b2)
    pl.loop(0, SC_K)(outer)

    for i in range(NBUF):
        pltpu.make_async_copy(rows_v[i], acc_sh.at[dst_v[i]], semS[i]).wait()

    plsc.subcore_barrier()
    pltpu.sync_copy(acc_sh.at[pl.ds(r0, RPS)],
                    out_hbm.at[c].at[pl.ds(r0, RPS)])


BN = 1000


def _tc1_body(x_ref, w_ref, deg_ref, g_ref, norm_ref):
    nb = lax.rsqrt(deg_ref[0] + deg_ref[1] + 1.0)
    norm_ref[...] = nb
    g = jnp.dot(x_ref[...], w_ref[...], preferred_element_type=jnp.float32) * nb
    g_ref[0] = g[:, :HALF]
    g_ref[1] = g[:, HALF:]


def _tc_mid_body(s_ref, g_ref, norm_ref, b_ref, w_ref, go_ref):
    nb = norm_ref[...]
    h = jnp.concatenate([s_ref[0] + g_ref[0], s_ref[1] + g_ref[1]], axis=-1)
    h = jnp.maximum(h * nb + b_ref[...], 0.0)
    g = jnp.dot(h, w_ref[...], preferred_element_type=jnp.float32) * nb
    go_ref[0] = g[:, :HALF]
    go_ref[1] = g[:, HALF:]


def _tc_final_body(s_ref, g_ref, norm_ref, b_ref, o_ref):
    h = jnp.concatenate([s_ref[0] + g_ref[0], s_ref[1] + g_ref[1]], axis=-1)
    o_ref[...] = h * norm_ref[...] + b_ref[...]


def _tc1(x, w1, deg):
    return pl.pallas_call(
        _tc1_body,
        grid=(N // BN,),
        in_specs=[
            pl.BlockSpec((BN, D_IN), lambda i: (i, 0)),
            pl.BlockSpec((D_IN, H), lambda i: (0, 0)),
            pl.BlockSpec((2, BN, 1), lambda i: (0, i, 0)),
        ],
        out_specs=[
            pl.BlockSpec((2, BN, HALF), lambda i: (0, i, 0)),
            pl.BlockSpec((BN, 1), lambda i: (i, 0)),
        ],
        out_shape=[
            jax.ShapeDtypeStruct((2, N, HALF), jnp.float32),
            jax.ShapeDtypeStruct((N, 1), jnp.float32),
        ],
    )(x, w1, deg)


def _tc_mid(sh, gh, norm, b, w):
    return pl.pallas_call(
        _tc_mid_body,
        grid=(N // BN,),
        in_specs=[
            pl.BlockSpec((2, BN, HALF), lambda i: (0, i, 0)),
            pl.BlockSpec((2, BN, HALF), lambda i: (0, i, 0)),
            pl.BlockSpec((BN, 1), lambda i: (i, 0)),
            pl.BlockSpec((1, H), lambda i: (0, 0)),
            pl.BlockSpec((H, H), lambda i: (0, 0)),
        ],
        out_specs=pl.BlockSpec((2, BN, HALF), lambda i: (0, i, 0)),
        out_shape=jax.ShapeDtypeStruct((2, N, HALF), jnp.float32),
    )(sh, gh, norm, b, w)


def _tc_final(sh, gh, norm, b):
    return pl.pallas_call(
        _tc_final_body,
        grid=(N // BN,),
        in_specs=[
            pl.BlockSpec((2, BN, HALF), lambda i: (0, i, 0)),
            pl.BlockSpec((2, BN, HALF), lambda i: (0, i, 0)),
            pl.BlockSpec((BN, 1), lambda i: (i, 0)),
            pl.BlockSpec((1, H), lambda i: (0, 0)),
        ],
        out_specs=pl.BlockSpec((BN, H), lambda i: (i, 0)),
        out_shape=jax.ShapeDtypeStruct((N, H), jnp.float32),
    )(sh, gh, norm, b)


def kernel(x, edge_index, W1, b1, W2, b2, W3, b3):
    src = edge_index[0]
    dst = edge_index[1]
    deg = _deg_kernel(dst).reshape(2, N_PAD, 1)
    g1, norm = _tc1(x, W1, deg)
    s1 = _scatter_kernel(g1, src, dst)
    g2 = _tc_mid(s1, g1, norm, b1.reshape(1, H), W2)
    s2 = _scatter_kernel(g2, src, dst)
    g3 = _tc_mid(s2, g2, norm, b2.reshape(1, H), W3)
    s3 = _scatter_kernel(g3, src, dst)
    return _tc_final(s3, g3, norm, b3.reshape(1, H))


# speedup vs baseline: 1.1004x; 1.1004x over previous
import functools

import jax
import jax.numpy as jnp
from jax import lax
from jax.experimental import pallas as pl
from jax.experimental.pallas import tpu as pltpu
from jax.experimental.pallas import tpu_sc as plsc

N = 10000
E = 320000
D_IN = 128
H = 256
HALF = 128

N_PAD = 10240

_MESH = plsc.VectorSubcoreMesh(core_axis_name="c", subcore_axis_name="s")

DEG_CHUNK = 80
DEG_PER_SUB = E // 32


NBUF = 5
DEG_T = DEG_PER_SUB // DEG_CHUNK
DEG_K = DEG_T // NBUF


@functools.partial(
    pl.kernel,
    out_type=jax.ShapeDtypeStruct((2, N_PAD), jnp.float32),
    mesh=_MESH,
    scratch_types=(
        [pltpu.VMEM((DEG_CHUNK,), jnp.int32) for _ in range(NBUF)]
        + [pltpu.VMEM((DEG_CHUNK,), jnp.float32)]
        + [pltpu.VMEM((640,), jnp.float32)]
        + [pltpu.SemaphoreType.DMA for _ in range(2 * NBUF)]
        + [pltpu.VMEM_SHARED((N_PAD,), jnp.float32)]
    ),
)
def _deg_kernel(dst_hbm, out_hbm, *rest):
    idx_v = rest[:NBUF]
    ones_v = rest[NBUF]
    zeros_v = rest[NBUF + 1]
    semI = rest[NBUF + 2:2 * NBUF + 2]
    semS = rest[2 * NBUF + 2:3 * NBUF + 2]
    acc_sh = rest[3 * NBUF + 2]

    c = lax.axis_index("c")
    s = lax.axis_index("s")
    base = (c * 16 + s) * DEG_PER_SUB

    def fill(i):
        zeros_v[pl.ds(i * 16, 16)] = jnp.zeros((16,), jnp.float32)
    pl.loop(0, 40)(fill)

    def fill1(i):
        ones_v[pl.ds(i * 16, 16)] = jnp.ones((16,), jnp.float32)
    pl.loop(0, DEG_CHUNK // 16)(fill1)

    pltpu.sync_copy(zeros_v, acc_sh.at[pl.ds(s * 640, 640)])
    plsc.subcore_barrier()

    def issue_idx(t, b):
        pltpu.async_copy(
            dst_hbm.at[pl.ds(base + t * DEG_CHUNK, DEG_CHUNK)], idx_v[b], semI[b])

    for b in range(NBUF):
        issue_idx(b, b)

    def outer(k):
        for b in range(NBUF):
            t = k * NBUF + b
            pltpu.make_async_copy(
                dst_hbm.at[pl.ds(base + t * DEG_CHUNK, DEG_CHUNK)], idx_v[b],
                semI[b]).wait()
            pltpu.async_copy(ones_v, acc_sh.at[idx_v[b]], semS[b], add=True)
            b2 = (b - 1) % NBUF

            @pl.when(jnp.logical_and(t >= 1, t + (NBUF - 1) <= DEG_T - 1))
            def _():
                pltpu.make_async_copy(
                    ones_v, acc_sh.at[idx_v[b2]], semS[b2]).wait()
                issue_idx(t + NBUF - 1, b2)
    pl.loop(0, DEG_K)(outer)

    for i in range(NBUF):
        pltpu.make_async_copy(ones_v, acc_sh.at[idx_v[i]], semS[i]).wait()

    plsc.subcore_barrier()
    pltpu.sync_copy(acc_sh.at[pl.ds(s * 640, 640)],
                    out_hbm.at[c].at[pl.ds(s * 640, 640)])


SC_CHUNK = 40
SC_PER_SUB = E // 16


RPS = N_PAD // 16
SC_T = SC_PER_SUB // SC_CHUNK
SC_K = SC_T // NBUF


@functools.partial(
    pl.kernel,
    out_type=jax.ShapeDtypeStruct((2, N_PAD, HALF), jnp.float32),
    mesh=_MESH,
    scratch_types=(
        [pltpu.VMEM((SC_CHUNK, HALF), jnp.float32) for _ in range(NBUF)]
        + [pltpu.VMEM((SC_CHUNK,), jnp.int32) for _ in range(NBUF)]
        + [pltpu.VMEM((SC_PER_SUB,), jnp.int32)]
        + [pltpu.SemaphoreType.DMA for _ in range(3 * NBUF)]
        + [pltpu.VMEM_SHARED((N_PAD, HALF), jnp.float32)]
    ),
)
def _scatter_kernel(g_hbm, src_hbm, dst_hbm, out_hbm, *rest):
    rows_v = rest[:NBUF]
    dst_v = rest[NBUF:2 * NBUF]
    src_all = rest[2 * NBUF]
    semI = rest[2 * NBUF + 1:3 * NBUF + 1]
    semG = rest[3 * NBUF + 1:4 * NBUF + 1]
    semS = rest[4 * NBUF + 1:5 * NBUF + 1]
    acc_sh = rest[5 * NBUF + 1]

    c = lax.axis_index("c")
    s = lax.axis_index("s")
    base = s * SC_PER_SUB

    pltpu.async_copy(src_hbm.at[pl.ds(base, SC_PER_SUB)], src_all, semG[0])

    def zrow(i):
        def zcol(j):
            rows_v[0][i, pl.ds(j * 16, 16)] = jnp.zeros((16,), jnp.float32)
        pl.loop(0, HALF // 16)(zcol)
    pl.loop(0, SC_CHUNK)(zrow)

    r0 = s * RPS

    def zacc(k):
        pltpu.async_copy(rows_v[0], acc_sh.at[pl.ds(r0 + k * SC_CHUNK, SC_CHUNK)],
                         semS[0])
    pl.loop(0, RPS // SC_CHUNK)(zacc)

    def zwait(k):
        pltpu.make_async_copy(
            rows_v[0], acc_sh.at[pl.ds(r0 + k * SC_CHUNK, SC_CHUNK)], semS[0]).wait()
    pl.loop(0, RPS // SC_CHUNK)(zwait)
    pltpu.make_async_copy(src_hbm.at[pl.ds(base, SC_PER_SUB)], src_all,
                          semG[0]).wait()

    plsc.subcore_barrier()

    def issue_chunk(t, b):
        pltpu.async_copy(
            dst_hbm.at[pl.ds(base + t * SC_CHUNK, SC_CHUNK)], dst_v[b], semI[b])
        pltpu.async_copy(
            g_hbm.at[c].at[src_all.at[pl.ds(t * SC_CHUNK, SC_CHUNK)]],
            rows_v[b], semG[b])

    for b in range(NBUF):
        issue_chunk(b, b)

    def outer(k):
        for b in range(NBUF):
            t = k * NBUF + b
            pltpu.make_async_copy(
                dst_hbm.at[pl.ds(base + t * SC_CHUNK, SC_CHUNK)], dst_v[b],
                semI[b]).wait()
            pltpu.make_async_copy(
                g_hbm.at[c].at[src_all.at[pl.ds(t * SC_CHUNK, SC_CHUNK)]],
                rows_v[b], semG[b]).wait()
            pltpu.async_copy(rows_v[b], acc_sh.at[dst_v[b]], semS[b], add=True)
            b2 = (b - 1) % NBUF

            @pl.when(jnp.logical_and(t >= 1, t + (NBUF - 1) <= SC_T - 1))
            def _():
                pltpu.make_async_copy(
                    rows_v[b2], acc_sh.at[dst_v[b2]], semS[b2]).wait()
                issue_chunk(t + NBUF - 1, b2)
    pl.loop(0, SC_K)(outer)

    for i in range(NBUF):
        pltpu.make_async_copy(rows_v[i], acc_sh.at[dst_v[i]], semS[i]).wait()

    plsc.subcore_barrier()
    pltpu.sync_copy(acc_sh.at[pl.ds(r0, RPS)],
                    out_hbm.at[c].at[pl.ds(r0, RPS)])


BN = 1000


def _tc1_body(x_ref, w_ref, deg_ref, g_ref, norm_ref):
    nb = lax.rsqrt(deg_ref[0] + deg_ref[1] + 1.0)
    norm_ref[...] = nb
    g = jnp.dot(x_ref[...], w_ref[...], preferred_element_type=jnp.float32) * nb
    g_ref[0] = g[:, :HALF]
    g_ref[1] = g[:, HALF:]


def _tc_mid_body(s_ref, g_ref, norm_ref, b_ref, w_ref, go_ref):
    nb = norm_ref[...]
    h = jnp.concatenate([s_ref[0] + g_ref[0], s_ref[1] + g_ref[1]], axis=-1)
    h = jnp.maximum(h * nb + b_ref[...], 0.0)
    g = jnp.dot(h, w_ref[...], preferred_element_type=jnp.float32) * nb
    go_ref[0] = g[:, :HALF]
    go_ref[1] = g[:, HALF:]


def _tc_final_body(s_ref, g_ref, norm_ref, b_ref, o_ref):
    h = jnp.concatenate([s_ref[0] + g_ref[0], s_ref[1] + g_ref[1]], axis=-1)
    o_ref[...] = h * norm_ref[...] + b_ref[...]


def _tc1(x, w1, deg):
    return pl.pallas_call(
        _tc1_body,
        grid=(N // BN,),
        in_specs=[
            pl.BlockSpec((BN, D_IN), lambda i: (i, 0)),
            pl.BlockSpec((D_IN, H), lambda i: (0, 0)),
            pl.BlockSpec((2, BN, 1), lambda i: (0, i, 0)),
        ],
        out_specs=[
            pl.BlockSpec((2, BN, HALF), lambda i: (0, i, 0)),
            pl.BlockSpec((BN, 1), lambda i: (i, 0)),
        ],
        out_shape=[
            jax.ShapeDtypeStruct((2, N, HALF), jnp.float32),
            jax.ShapeDtypeStruct((N, 1), jnp.float32),
        ],
    )(x, w1, deg)


def _tc_mid(sh, gh, norm, b, w):
    return pl.pallas_call(
        _tc_mid_body,
        grid=(N // BN,),
        in_specs=[
            pl.BlockSpec((2, BN, HALF), lambda i: (0, i, 0)),
            pl.BlockSpec((2, BN, HALF), lambda i: (0, i, 0)),
            pl.BlockSpec((BN, 1), lambda i: (i, 0)),
            pl.BlockSpec((1, H), lambda i: (0, 0)),
            pl.BlockSpec((H, H), lambda i: (0, 0)),
        ],
        out_specs=pl.BlockSpec((2, BN, HALF), lambda i: (0, i, 0)),
        out_shape=jax.ShapeDtypeStruct((2, N, HALF), jnp.float32),
    )(sh, gh, norm, b, w)


def _tc_final(sh, gh, norm, b):
    return pl.pallas_call(
        _tc_final_body,
        grid=(N // BN,),
        in_specs=[
            pl.BlockSpec((2, BN, HALF), lambda i: (0, i, 0)),
            pl.BlockSpec((2, BN, HALF), lambda i: (0, i, 0)),
            pl.BlockSpec((BN, 1), lambda i: (i, 0)),
            pl.BlockSpec((1, H), lambda i: (0, 0)),
        ],
        out_specs=pl.BlockSpec((BN, H), lambda i: (i, 0)),
        out_shape=jax.ShapeDtypeStruct((N, H), jnp.float32),
    )(sh, gh, norm, b)


def kernel(x, edge_index, W1, b1, W2, b2, W3, b3):
    src = edge_index[0]
    dst = edge_index[1]
    deg = _deg_kernel(dst).reshape(2, N_PAD, 1)
    g1, norm = _tc1(x, W1, deg)
    s1 = _scatter_kernel(g1, src, dst)
    g2 = _tc_mid(s1, g1, norm, b1.reshape(1, H), W2)
    s2 = _scatter_kernel(g2, src, dst)
    g3 = _tc_mid(s2, g2, norm, b2.reshape(1, H), W3)
    s3 = _scatter_kernel(g3, src, dst)
    return _tc_final(s3, g3, norm, b3.reshape(1, H))
